# baseline (speedup 1.0000x reference)
import functools
import math

import jax
import jax.numpy as jnp
from jax.experimental import pallas as pl
from jax.experimental.pallas import tpu as pltpu

_INV_SQRT2 = 1.0 / math.sqrt(2.0)


def _gelu_exact_f32(h):
    return 0.5 * h * (1.0 + jax.lax.erf(h * jnp.float32(_INV_SQRT2)))


def _ffn_kernel(x_ref, w1_ref, b1_ref, w2_ref, b2_ref, o_ref, acc_ref,
                *, subtiles):
    k = pl.program_id(1)
    tm = x_ref.shape[0]
    sub = tm // subtiles

    @pl.when(k == 0)
    def _():
        acc_ref[...] = jnp.zeros_like(acc_ref)

    for s in range(subtiles):
        rows = pl.ds(s * sub, sub)
        h = jnp.dot(x_ref[rows, :], w1_ref[...],
                    preferred_element_type=jnp.float32)
        g = _gelu_exact_f32(h + b1_ref[...])
        acc_ref[rows, :] += jnp.dot(g, w2_ref[...],
                                    preferred_element_type=jnp.float32)

    @pl.when(k == pl.num_programs(1) - 1)
    def _():
        o_ref[...] = acc_ref[...] + b2_ref[...]


@functools.partial(jax.jit, static_argnames=("tm", "th", "subtiles"))
def _mlp_forward(x, w1, b1, w2, b2, *, tm=1024, th=512, subtiles=2):
    B, N, in_feat = x.shape
    hid = w1.shape[1]
    out_feat = w2.shape[1]
    M = B * N

    x2 = x.reshape(M, in_feat)
    b1_2d = b1.reshape(1, hid)
    b2_2d = b2.reshape(1, out_feat)

    single = pl.Buffered(1)
    grid = (pl.cdiv(M, tm), hid // th)

    cost = pl.CostEstimate(
        flops=int(2 * M * (in_feat * hid + hid * out_feat)),
        transcendentals=int(M * hid),
        bytes_accessed=int(M * in_feat * 4
                           + (in_feat * hid + hid * out_feat) * 4
                           + (hid + out_feat) * 4
                           + M * out_feat * 4),
    )

    y2 = pl.pallas_call(
        functools.partial(_ffn_kernel, subtiles=subtiles),
        out_shape=jax.ShapeDtypeStruct((M, out_feat), jnp.float32),
        grid_spec=pltpu.PrefetchScalarGridSpec(
            num_scalar_prefetch=0,
            grid=grid,
            in_specs=[
                pl.BlockSpec((tm, in_feat), lambda i, k: (i, 0),
                             pipeline_mode=single),
                pl.BlockSpec((in_feat, th), lambda i, k: (0, k)),
                pl.BlockSpec((1, th), lambda i, k: (0, k)),
                pl.BlockSpec((th, out_feat), lambda i, k: (k, 0)),
                pl.BlockSpec((1, out_feat), lambda i, k: (0, 0),
                             pipeline_mode=single),
            ],
            out_specs=pl.BlockSpec((tm, out_feat), lambda i, k: (i, 0)),
            scratch_shapes=[pltpu.VMEM((tm, out_feat), jnp.float32)],
        ),
        compiler_params=pltpu.CompilerParams(
            dimension_semantics=("parallel", "arbitrary"),
            vmem_limit_bytes=96 * 1024 * 1024,
        ),
        cost_estimate=cost,
    )(x2, w1, b1_2d, w2, b2_2d)

    return y2.reshape(B, N, out_feat)


def kernel(x, w1, b1, w2, b2):
    return _mlp_forward(x, w1, b1, w2, b2)


# speedup vs baseline: 1.1650x; 1.1650x over previous
import functools
import math

import jax
import jax.numpy as jnp
from jax.experimental import pallas as pl
from jax.experimental.pallas import tpu as pltpu

_INV_SQRT2 = 1.0 / math.sqrt(2.0)


def _gelu_exact_f32(h):
    return 0.5 * h * (1.0 + jax.lax.erf(h * jnp.float32(_INV_SQRT2)))


def _ffn_kernel(x_ref, w1_ref, b1_ref, w2_ref, b2_ref, o_ref, acc_ref,
                *, subtiles):
    k = pl.program_id(1)
    tm = x_ref.shape[0]
    sub = tm // subtiles

    @pl.when(k == 0)
    def _():
        acc_ref[...] = jnp.zeros_like(acc_ref)

    for s in range(subtiles):
        rows = pl.ds(s * sub, sub)
        h = jnp.dot(x_ref[rows, :], w1_ref[...],
                    preferred_element_type=jnp.float32)
        g = _gelu_exact_f32(h + b1_ref[...])
        acc_ref[rows, :] += jnp.dot(g, w2_ref[...],
                                    preferred_element_type=jnp.float32)

    @pl.when(k == pl.num_programs(1) - 1)
    def _():
        o_ref[...] = acc_ref[...] + b2_ref[...]


@functools.partial(jax.jit, static_argnames=("tm", "th", "subtiles"))
def _mlp_forward(x, w1, b1, w2, b2, *, tm=2048, th=512, subtiles=2):
    B, N, in_feat = x.shape
    hid = w1.shape[1]
    out_feat = w2.shape[1]
    M = B * N

    x2 = x.reshape(M, in_feat)
    b1_2d = b1.reshape(1, hid)
    b2_2d = b2.reshape(1, out_feat)

    single = pl.Buffered(1)
    grid = (pl.cdiv(M, tm), hid // th)

    cost = pl.CostEstimate(
        flops=int(2 * M * (in_feat * hid + hid * out_feat)),
        transcendentals=int(M * hid),
        bytes_accessed=int(M * in_feat * 4
                           + (in_feat * hid + hid * out_feat) * 4
                           + (hid + out_feat) * 4
                           + M * out_feat * 4),
    )

    y2 = pl.pallas_call(
        functools.partial(_ffn_kernel, subtiles=subtiles),
        out_shape=jax.ShapeDtypeStruct((M, out_feat), jnp.float32),
        grid_spec=pltpu.PrefetchScalarGridSpec(
            num_scalar_prefetch=0,
            grid=grid,
            in_specs=[
                pl.BlockSpec((tm, in_feat), lambda i, k: (i, 0)),
                pl.BlockSpec((in_feat, th), lambda i, k: (0, k)),
                pl.BlockSpec((1, th), lambda i, k: (0, k)),
                pl.BlockSpec((th, out_feat), lambda i, k: (k, 0)),
                pl.BlockSpec((1, out_feat), lambda i, k: (0, 0),
                             pipeline_mode=single),
            ],
            out_specs=pl.BlockSpec((tm, out_feat), lambda i, k: (i, 0)),
            scratch_shapes=[pltpu.VMEM((tm, out_feat), jnp.float32)],
        ),
        compiler_params=pltpu.CompilerParams(
            dimension_semantics=("parallel", "arbitrary"),
            vmem_limit_bytes=96 * 1024 * 1024,
        ),
        cost_estimate=cost,
    )(x2, w1, b1_2d, w2, b2_2d)

    return y2.reshape(B, N, out_feat)


def kernel(x, w1, b1, w2, b2):
    return _mlp_forward(x, w1, b1, w2, b2)
